# Initial kernel scaffold; baseline (speedup 1.0000x reference)
import jax
import jax.numpy as jnp
from jax.experimental import pallas as pl


def kernel(q, k, v, W_perf):
    raise NotImplementedError("write your pallas kernel here")


# speedup vs baseline: 9.4716x; 9.4716x over previous
import jax
import jax.numpy as jnp
import numpy as np
from jax.experimental import pallas as pl
from jax.experimental.pallas import tpu as pltpu

B, H, S, DH = 1, 12, 2048, 64
M = 64
TOPK = 128
QBLK = 256


def _phi_block(x, w):
    proj = jax.lax.dot_general(
        x, w, (((1,), (0,)), ((), ())), preferred_element_type=jnp.float32)
    sq_norm = jnp.sum(x * x, axis=-1, keepdims=True) / 2.0
    stab = jnp.max(proj, axis=-1, keepdims=True)
    return jnp.exp(proj - sq_norm - stab) + 1e-6


def _kp_kernel(k_ref, w_ref, kp_ref):
    x = k_ref[0, 0] * (DH ** -0.25)
    kp_ref[0, 0] = _phi_block(x, w_ref[...])


def _attn_kernel(q_ref, k_ref, v_ref, kp_ref, w_ref, out_ref):
    q = q_ref[0, 0]
    k = k_ref[0, 0]
    v = v_ref[0, 0]
    kp = kp_ref[0, 0]

    qp = _phi_block(q * (DH ** -0.25), w_ref[...])
    est = jax.lax.dot_general(
        qp, kp, (((1,), (1,)), ((), ())), preferred_element_type=jnp.float32)

    bits = jax.lax.bitcast_convert_type(est, jnp.int32)
    acc = jnp.zeros((QBLK, 1), jnp.int32)
    for b in range(30, -1, -1):
        cand = acc | jnp.int32(1 << b)
        cnt = jnp.sum((bits >= cand).astype(jnp.int32), axis=1, keepdims=True)
        acc = jnp.where(cnt >= TOPK, cand, acc)
    mask = bits >= acc

    scores = jax.lax.dot_general(
        q, k, (((1,), (1,)), ((), ())),
        preferred_element_type=jnp.float32) * (1.0 / np.sqrt(DH))
    ms = jnp.where(mask, scores, jnp.float32(-1e9))
    m = jnp.max(ms, axis=-1, keepdims=True)
    p = jnp.exp(ms - m)
    probs = p / jnp.sum(p, axis=-1, keepdims=True)
    out_ref[0] = jax.lax.dot_general(
        probs, v, (((1,), (0,)), ((), ())), preferred_element_type=jnp.float32)


def _out_transpose(out_hsd):
    return jnp.transpose(out_hsd, (1, 0, 2)).reshape(B, S, H * DH)


@jax.jit
def kernel(q, k, v, W_perf):
    kp = pl.pallas_call(
        _kp_kernel,
        grid=(H,),
        in_specs=[
            pl.BlockSpec((1, 1, S, DH), lambda h: (0, h, 0, 0)),
            pl.BlockSpec((DH, M), lambda h: (0, 0)),
        ],
        out_specs=pl.BlockSpec((1, 1, S, M), lambda h: (0, h, 0, 0)),
        out_shape=jax.ShapeDtypeStruct((B, H, S, M), jnp.float32),
    )(k, W_perf)

    out = pl.pallas_call(
        _attn_kernel,
        grid=(H, S // QBLK),
        in_specs=[
            pl.BlockSpec((1, 1, QBLK, DH), lambda h, i: (0, h, i, 0)),
            pl.BlockSpec((1, 1, S, DH), lambda h, i: (0, h, 0, 0)),
            pl.BlockSpec((1, 1, S, DH), lambda h, i: (0, h, 0, 0)),
            pl.BlockSpec((1, 1, S, M), lambda h, i: (0, h, 0, 0)),
            pl.BlockSpec((DH, M), lambda h, i: (0, 0)),
        ],
        out_specs=pl.BlockSpec((1, QBLK, DH), lambda h, i: (h, i, 0)),
        out_shape=jax.ShapeDtypeStruct((H, S, DH), jnp.float32),
    )(q, k, v, kp, W_perf)
    return _out_transpose(out)


# speedup vs baseline: 11.3164x; 1.1948x over previous
import jax
import jax.numpy as jnp
import numpy as np
from jax.experimental import pallas as pl
from jax.experimental.pallas import tpu as pltpu

B, H, S, DH = 1, 12, 2048, 64
M = 64
TOPK = 128
QBLK = 256


def _phi_block(x, w):
    proj = jax.lax.dot_general(
        x, w, (((1,), (0,)), ((), ())), preferred_element_type=jnp.float32)
    sq_norm = jnp.sum(x * x, axis=-1, keepdims=True) / 2.0
    stab = jnp.max(proj, axis=-1, keepdims=True)
    return jnp.exp(proj - sq_norm - stab) + 1e-6


def _kp_kernel(k_ref, w_ref, kp_ref):
    x = k_ref[0, 0] * (DH ** -0.25)
    kp_ref[0, 0] = _phi_block(x, w_ref[...])


def _bits16_to_bf(x_i32):
    return jax.lax.bitcast_convert_type(
        x_i32 << 16, jnp.float32).astype(jnp.bfloat16)


def _count_ge(mask_src, one, zero, acc_dtype):
    m = jnp.where(mask_src, one, zero)
    r = S
    while r > 16:
        r //= 2
        m = m[:r] + m[r:]
    mf = m.astype(jnp.float32)
    return jnp.sum(mf, axis=0, keepdims=True)


def _attn_kernel(q_ref, k_ref, v_ref, kp_ref, w_ref, out_ref):
    q = q_ref[0, 0]
    k = k_ref[0, 0]
    v = v_ref[0, 0]
    kp = kp_ref[0, 0]

    qp = _phi_block(q * (DH ** -0.25), w_ref[...])
    est_t = jax.lax.dot_general(
        kp, qp, (((1,), (1,)), ((), ())),
        preferred_element_type=jnp.float32)

    bits = jax.lax.bitcast_convert_type(est_t, jnp.int32)
    one_bf = jnp.bfloat16(1.0)
    zero_bf = jnp.bfloat16(0.0)
    ktop = jnp.float32(TOPK)

    hi16 = jax.lax.shift_right_logical(bits, 16)
    est_bf = _bits16_to_bf(hi16)
    acc16 = jnp.zeros((1, QBLK), jnp.int32)
    for b in range(14, -1, -1):
        cand16 = acc16 | jnp.int32(1 << b)
        cnt = _count_ge(est_bf >= _bits16_to_bf(cand16), one_bf, zero_bf,
                        jnp.bfloat16)
        acc16 = jnp.where(cnt >= ktop, cand16, acc16)

    cnt_gt = _count_ge(est_bf >= _bits16_to_bf(acc16 + 1), one_bf, zero_bf,
                       jnp.bfloat16)
    k2 = ktop - cnt_gt

    in_bucket = hi16 == acc16
    w_pat = ((bits & jnp.int32(0xFFFC)) >> 2) + jnp.int32(0x2000)
    w_bf = jnp.where(in_bucket, _bits16_to_bf(w_pat), zero_bf)
    accy = jnp.zeros((1, QBLK), jnp.int32)
    for b in range(13, -1, -1):
        candy = accy | jnp.int32(1 << b)
        cnt = _count_ge(
            w_bf >= _bits16_to_bf(candy + jnp.int32(0x2000)),
            one_bf, zero_bf, jnp.bfloat16)
        accy = jnp.where(cnt >= k2, candy, accy)

    acc = (acc16 << 16) | (accy << 2)
    for b in (1, 0):
        cand = acc | jnp.int32(1 << b)
        cnt = _count_ge(bits >= cand, jnp.float32(1.0), jnp.float32(0.0),
                        jnp.float32)
        acc = jnp.where(cnt >= ktop, cand, acc)

    scores_t = jax.lax.dot_general(
        k, q, (((1,), (1,)), ((), ())),
        preferred_element_type=jnp.float32) * (1.0 / np.sqrt(DH))
    ms = jnp.where(bits >= acc, scores_t, jnp.float32(-1e9))
    mx = jnp.max(ms, axis=0, keepdims=True)
    p = jnp.exp(ms - mx)
    probs = p / jnp.sum(p, axis=0, keepdims=True)
    out_ref[0] = jax.lax.dot_general(
        probs, v, (((0,), (0,)), ((), ())), preferred_element_type=jnp.float32)


@jax.jit
def kernel(q, k, v, W_perf):
    kp = pl.pallas_call(
        _kp_kernel,
        grid=(H,),
        in_specs=[
            pl.BlockSpec((1, 1, S, DH), lambda h: (0, h, 0, 0)),
            pl.BlockSpec((DH, M), lambda h: (0, 0)),
        ],
        out_specs=pl.BlockSpec((1, 1, S, M), lambda h: (0, h, 0, 0)),
        out_shape=jax.ShapeDtypeStruct((B, H, S, M), jnp.float32),
    )(k, W_perf)

    out = pl.pallas_call(
        _attn_kernel,
        grid=(H, S // QBLK),
        in_specs=[
            pl.BlockSpec((1, 1, QBLK, DH), lambda h, i: (0, h, i, 0)),
            pl.BlockSpec((1, 1, S, DH), lambda h, i: (0, h, 0, 0)),
            pl.BlockSpec((1, 1, S, DH), lambda h, i: (0, h, 0, 0)),
            pl.BlockSpec((1, 1, S, M), lambda h, i: (0, h, 0, 0)),
            pl.BlockSpec((DH, M), lambda h, i: (0, 0)),
        ],
        out_specs=pl.BlockSpec((1, QBLK, DH), lambda h, i: (h, i, 0)),
        out_shape=jax.ShapeDtypeStruct((H, S, DH), jnp.float32),
    )(q, k, v, kp, W_perf)
    return jnp.transpose(out, (1, 0, 2)).reshape(B, S, H * DH)


# speedup vs baseline: 11.8508x; 1.0472x over previous
import jax
import jax.numpy as jnp
import numpy as np
from jax.experimental import pallas as pl
from jax.experimental.pallas import tpu as pltpu

B, H, S, DH = 1, 12, 2048, 64
M = 64
TOPK = 128
QBLK = 512


def _phi_block(x, w):
    proj = jax.lax.dot_general(
        x, w, (((1,), (0,)), ((), ())), preferred_element_type=jnp.float32)
    sq_norm = jnp.sum(x * x, axis=-1, keepdims=True) / 2.0
    stab = jnp.max(proj, axis=-1, keepdims=True)
    return jnp.exp(proj - sq_norm - stab) + 1e-6


def _kp_kernel(k_ref, w_ref, kp_ref):
    x = k_ref[0, 0] * (DH ** -0.25)
    kp_ref[0, 0] = _phi_block(x, w_ref[...])


def _bits16_to_bf(x_i32):
    return jax.lax.bitcast_convert_type(
        x_i32 << 16, jnp.float32).astype(jnp.bfloat16)


def _count_ge(mask_src, one, zero, acc_dtype):
    m = jnp.where(mask_src, one, zero)
    r = S
    while r > 16:
        r //= 2
        m = m[:r] + m[r:]
    mf = m.astype(jnp.float32)
    return jnp.sum(mf, axis=0, keepdims=True)


def _attn_kernel(q_ref, k_ref, v_ref, kp_ref, w_ref, out_ref):
    q = q_ref[0, 0]
    k = k_ref[0, 0]
    v = v_ref[0, 0]
    kp = kp_ref[0, 0]

    qp = _phi_block(q * (DH ** -0.25), w_ref[...])
    est_t = jax.lax.dot_general(
        kp, qp, (((1,), (1,)), ((), ())),
        preferred_element_type=jnp.float32)

    bits = jax.lax.bitcast_convert_type(est_t, jnp.int32)
    one_bf = jnp.bfloat16(1.0)
    zero_bf = jnp.bfloat16(0.0)
    ktop = jnp.float32(TOPK)

    hi16 = jax.lax.shift_right_logical(bits, 16)
    est_bf = _bits16_to_bf(hi16)
    acc16 = jnp.zeros((1, QBLK), jnp.int32)
    for b in range(14, -1, -1):
        cand16 = acc16 | jnp.int32(1 << b)
        cnt = _count_ge(est_bf >= _bits16_to_bf(cand16), one_bf, zero_bf,
                        jnp.bfloat16)
        acc16 = jnp.where(cnt >= ktop, cand16, acc16)

    cnt_gt = _count_ge(est_bf >= _bits16_to_bf(acc16 + 1), one_bf, zero_bf,
                       jnp.bfloat16)
    k2 = ktop - cnt_gt

    in_bucket = hi16 == acc16
    w_pat = ((bits & jnp.int32(0xFFFC)) >> 2) + jnp.int32(0x2000)
    w_bf = jnp.where(in_bucket, _bits16_to_bf(w_pat), zero_bf)
    accy = jnp.zeros((1, QBLK), jnp.int32)
    for b in range(13, -1, -1):
        candy = accy | jnp.int32(1 << b)
        cnt = _count_ge(
            w_bf >= _bits16_to_bf(candy + jnp.int32(0x2000)),
            one_bf, zero_bf, jnp.bfloat16)
        accy = jnp.where(cnt >= k2, candy, accy)

    acc = (acc16 << 16) | (accy << 2)
    for b in (1, 0):
        cand = acc | jnp.int32(1 << b)
        cnt = _count_ge(bits >= cand, jnp.float32(1.0), jnp.float32(0.0),
                        jnp.float32)
        acc = jnp.where(cnt >= ktop, cand, acc)

    scores_t = jax.lax.dot_general(
        k, q, (((1,), (1,)), ((), ())),
        preferred_element_type=jnp.float32) * (1.0 / np.sqrt(DH))
    ms = jnp.where(bits >= acc, scores_t, jnp.float32(-1e9))
    mx = jnp.max(ms, axis=0, keepdims=True)
    p = jnp.exp(ms - mx)
    probs = p / jnp.sum(p, axis=0, keepdims=True)
    out_ref[0] = jax.lax.dot_general(
        probs, v, (((0,), (0,)), ((), ())), preferred_element_type=jnp.float32)


@jax.jit
def kernel(q, k, v, W_perf):
    kp = pl.pallas_call(
        _kp_kernel,
        grid=(H,),
        in_specs=[
            pl.BlockSpec((1, 1, S, DH), lambda h: (0, h, 0, 0)),
            pl.BlockSpec((DH, M), lambda h: (0, 0)),
        ],
        out_specs=pl.BlockSpec((1, 1, S, M), lambda h: (0, h, 0, 0)),
        out_shape=jax.ShapeDtypeStruct((B, H, S, M), jnp.float32),
    )(k, W_perf)

    out = pl.pallas_call(
        _attn_kernel,
        grid=(H, S // QBLK),
        in_specs=[
            pl.BlockSpec((1, 1, QBLK, DH), lambda h, i: (0, h, i, 0)),
            pl.BlockSpec((1, 1, S, DH), lambda h, i: (0, h, 0, 0)),
            pl.BlockSpec((1, 1, S, DH), lambda h, i: (0, h, 0, 0)),
            pl.BlockSpec((1, 1, S, M), lambda h, i: (0, h, 0, 0)),
            pl.BlockSpec((DH, M), lambda h, i: (0, 0)),
        ],
        out_specs=pl.BlockSpec((1, QBLK, DH), lambda h, i: (h, i, 0)),
        out_shape=jax.ShapeDtypeStruct((H, S, DH), jnp.float32),
    )(q, k, v, kp, W_perf)
    return jnp.transpose(out, (1, 0, 2)).reshape(B, S, H * DH)


# speedup vs baseline: 13.4062x; 1.1312x over previous
import jax
import jax.numpy as jnp
import numpy as np
from jax.experimental import pallas as pl
from jax.experimental.pallas import tpu as pltpu

B, H, S, DH = 1, 12, 2048, 64
M = 64
TOPK = 128
QBLK = 512


def _phi_block(x, w):
    proj = jax.lax.dot_general(
        x, w, (((1,), (0,)), ((), ())), preferred_element_type=jnp.float32)
    sq_norm = jnp.sum(x * x, axis=-1, keepdims=True) / 2.0
    stab = jnp.max(proj, axis=-1, keepdims=True)
    return jnp.exp(proj - sq_norm - stab) + 1e-6


def _kp_kernel(k_ref, w_ref, kp_ref):
    x = k_ref[0, 0] * (DH ** -0.25)
    kp_ref[0, 0] = _phi_block(x, w_ref[...])


def _bits16_to_bf(x_i32):
    return jax.lax.bitcast_convert_type(
        x_i32 << 16, jnp.float32).astype(jnp.bfloat16)


_CHUNK = 128


def _count_ge(x, cand, one, zero):
    acc = None
    for i in range(S // _CHUNK):
        m = jnp.where(x[i * _CHUNK:(i + 1) * _CHUNK] >= cand, one, zero)
        acc = m if acc is None else acc + m
    return jnp.sum(acc.astype(jnp.float32), axis=0, keepdims=True)


def _attn_kernel(q_ref, k_ref, v_ref, kp_ref, w_ref, out_ref):
    q = q_ref[0, 0]
    k = k_ref[0, 0]
    v = v_ref[0, 0]
    kp = kp_ref[0, 0]

    qp = _phi_block(q * (DH ** -0.25), w_ref[...])
    est_t = jax.lax.dot_general(
        kp, qp, (((1,), (1,)), ((), ())),
        preferred_element_type=jnp.float32)

    bits = jax.lax.bitcast_convert_type(est_t, jnp.int32)
    one_bf = jnp.bfloat16(1.0)
    zero_bf = jnp.bfloat16(0.0)
    ktop = jnp.float32(TOPK)

    hi16 = jax.lax.shift_right_logical(bits, 16)
    est_bf = _bits16_to_bf(hi16)
    lo16 = jnp.full((1, QBLK), 0x2E80, jnp.int32)
    hi16b = jnp.full((1, QBLK), 0x4281, jnp.int32)
    for _ in range(13):
        mid = (lo16 + hi16b) >> 1
        cnt = _count_ge(est_bf, _bits16_to_bf(mid), one_bf, zero_bf)
        ge = cnt >= ktop
        lo16 = jnp.where(ge, mid, lo16)
        hi16b = jnp.where(ge, hi16b, mid)
    acc16 = lo16

    cnt_gt = _count_ge(est_bf, _bits16_to_bf(acc16 + 1), one_bf, zero_bf)
    k2 = ktop - cnt_gt

    in_bucket = hi16 == acc16
    w_pat = ((bits & jnp.int32(0xFFFC)) >> 2) + jnp.int32(0x2000)
    w_bf = jnp.where(in_bucket, _bits16_to_bf(w_pat), zero_bf)
    accy = jnp.zeros((1, QBLK), jnp.int32)
    for b in range(13, -1, -1):
        candy = accy | jnp.int32(1 << b)
        cnt = _count_ge(w_bf, _bits16_to_bf(candy + jnp.int32(0x2000)),
                        one_bf, zero_bf)
        accy = jnp.where(cnt >= k2, candy, accy)

    acc = (acc16 << 16) | (accy << 2)
    for b in (1, 0):
        cand = acc | jnp.int32(1 << b)
        cnt = _count_ge(bits, cand, jnp.float32(1.0), jnp.float32(0.0))
        acc = jnp.where(cnt >= ktop, cand, acc)

    scores_t = jax.lax.dot_general(
        k, q, (((1,), (1,)), ((), ())),
        preferred_element_type=jnp.float32) * (1.0 / np.sqrt(DH))
    ms = jnp.where(bits >= acc, scores_t, jnp.float32(-1e9))
    mx = jnp.max(ms, axis=0, keepdims=True)
    p = jnp.exp(ms - mx)
    probs = p / jnp.sum(p, axis=0, keepdims=True)
    out_ref[0] = jax.lax.dot_general(
        probs, v, (((0,), (0,)), ((), ())), preferred_element_type=jnp.float32)


@jax.jit
def kernel(q, k, v, W_perf):
    kp = pl.pallas_call(
        _kp_kernel,
        grid=(H,),
        in_specs=[
            pl.BlockSpec((1, 1, S, DH), lambda h: (0, h, 0, 0)),
            pl.BlockSpec((DH, M), lambda h: (0, 0)),
        ],
        out_specs=pl.BlockSpec((1, 1, S, M), lambda h: (0, h, 0, 0)),
        out_shape=jax.ShapeDtypeStruct((B, H, S, M), jnp.float32),
    )(k, W_perf)

    out = pl.pallas_call(
        _attn_kernel,
        grid=(H, S // QBLK),
        in_specs=[
            pl.BlockSpec((1, 1, QBLK, DH), lambda h, i: (0, h, i, 0)),
            pl.BlockSpec((1, 1, S, DH), lambda h, i: (0, h, 0, 0)),
            pl.BlockSpec((1, 1, S, DH), lambda h, i: (0, h, 0, 0)),
            pl.BlockSpec((1, 1, S, M), lambda h, i: (0, h, 0, 0)),
            pl.BlockSpec((DH, M), lambda h, i: (0, 0)),
        ],
        out_specs=pl.BlockSpec((1, QBLK, DH), lambda h, i: (h, i, 0)),
        out_shape=jax.ShapeDtypeStruct((H, S, DH), jnp.float32),
    )(q, k, v, kp, W_perf)
    return jnp.transpose(out, (1, 0, 2)).reshape(B, S, H * DH)


# speedup vs baseline: 14.2132x; 1.0602x over previous
import jax
import jax.numpy as jnp
import numpy as np
from jax.experimental import pallas as pl
from jax.experimental.pallas import tpu as pltpu

B, H, S, DH = 1, 12, 2048, 64
M = 64
TOPK = 128
QBLK = 512


def _phi_block(x, w):
    proj = jax.lax.dot_general(
        x, w, (((1,), (0,)), ((), ())), preferred_element_type=jnp.float32)
    sq_norm = jnp.sum(x * x, axis=-1, keepdims=True) / 2.0
    stab = jnp.max(proj, axis=-1, keepdims=True)
    return jnp.exp(proj - sq_norm - stab) + 1e-6


def _kp_kernel(k_ref, w_ref, kp_ref):
    x = k_ref[0, 0] * (DH ** -0.25)
    kp_ref[0, 0] = _phi_block(x, w_ref[...])


def _bits16_to_bf(x_i32):
    return jax.lax.bitcast_convert_type(
        x_i32 << 16, jnp.float32).astype(jnp.bfloat16)


_CHUNK = 128


def _count_ge(x, cand, one, zero):
    acc = None
    for i in range(S // _CHUNK):
        m = jnp.where(x[i * _CHUNK:(i + 1) * _CHUNK] >= cand, one, zero)
        acc = m if acc is None else acc + m
    return jnp.sum(acc.astype(jnp.float32), axis=0, keepdims=True)


def _attn_kernel(q_ref, k_ref, v_ref, kp_ref, w_ref, out_ref):
    q = q_ref[0, 0]
    k = k_ref[0, 0]
    v = v_ref[0, 0]
    kp = kp_ref[0, 0]

    qp = _phi_block(q * (DH ** -0.25), w_ref[...])
    est_t = jax.lax.dot_general(
        kp, qp, (((1,), (1,)), ((), ())),
        preferred_element_type=jnp.float32)

    bits = jax.lax.bitcast_convert_type(est_t, jnp.int32)
    one_bf = jnp.bfloat16(1.0)
    zero_bf = jnp.bfloat16(0.0)
    ktop = jnp.float32(TOPK)

    hi16 = jax.lax.shift_right_logical(bits, 16)
    est_bf = _bits16_to_bf(hi16)
    lo16 = jnp.full((1, QBLK), 0x2E80, jnp.int32)
    hi16b = jnp.full((1, QBLK), 0x4281, jnp.int32)
    for _ in range(13):
        mid = (lo16 + hi16b) >> 1
        cnt = _count_ge(est_bf, _bits16_to_bf(mid), one_bf, zero_bf)
        ge = cnt >= ktop
        lo16 = jnp.where(ge, mid, lo16)
        hi16b = jnp.where(ge, hi16b, mid)
    acc16 = lo16

    cnt_gt = _count_ge(est_bf, _bits16_to_bf(acc16 + 1), one_bf, zero_bf)
    k2 = ktop - cnt_gt

    in_bucket = est_bf == _bits16_to_bf(acc16)
    w_pat = ((bits & jnp.int32(0xFFFC)) >> 2) + jnp.int32(0x2000)
    w_bf = jnp.where(in_bucket, _bits16_to_bf(w_pat), zero_bf)
    accy = jnp.zeros((1, QBLK), jnp.int32)
    for b in range(13, -1, -1):
        candy = accy | jnp.int32(1 << b)
        cnt = _count_ge(w_bf, _bits16_to_bf(candy + jnp.int32(0x2000)),
                        one_bf, zero_bf)
        accy = jnp.where(cnt >= k2, candy, accy)

    acc = (acc16 << 16) | (accy << 2)
    for b in (1, 0):
        cand = acc | jnp.int32(1 << b)
        cnt = _count_ge(bits, cand, jnp.float32(1.0), jnp.float32(0.0))
        acc = jnp.where(cnt >= ktop, cand, acc)

    scores_t = jax.lax.dot_general(
        k, q, (((1,), (1,)), ((), ())),
        preferred_element_type=jnp.float32) * (1.0 / np.sqrt(DH))
    ms = jnp.where(bits >= acc, scores_t, jnp.float32(-1e9))
    mx = jnp.max(ms, axis=0, keepdims=True)
    p = jnp.exp(ms - mx)
    v_aug = jnp.concatenate([v, jnp.ones((S, 1), jnp.float32)], axis=1)
    ctx = jax.lax.dot_general(
        p, v_aug, (((0,), (0,)), ((), ())),
        preferred_element_type=jnp.float32)
    out_ref[0] = ctx[:, :DH] * (1.0 / ctx[:, DH:DH + 1])


@jax.jit
def kernel(q, k, v, W_perf):
    kp = pl.pallas_call(
        _kp_kernel,
        grid=(H,),
        in_specs=[
            pl.BlockSpec((1, 1, S, DH), lambda h: (0, h, 0, 0)),
            pl.BlockSpec((DH, M), lambda h: (0, 0)),
        ],
        out_specs=pl.BlockSpec((1, 1, S, M), lambda h: (0, h, 0, 0)),
        out_shape=jax.ShapeDtypeStruct((B, H, S, M), jnp.float32),
    )(k, W_perf)

    out = pl.pallas_call(
        _attn_kernel,
        grid=(H, S // QBLK),
        in_specs=[
            pl.BlockSpec((1, 1, QBLK, DH), lambda h, i: (0, h, i, 0)),
            pl.BlockSpec((1, 1, S, DH), lambda h, i: (0, h, 0, 0)),
            pl.BlockSpec((1, 1, S, DH), lambda h, i: (0, h, 0, 0)),
            pl.BlockSpec((1, 1, S, M), lambda h, i: (0, h, 0, 0)),
            pl.BlockSpec((DH, M), lambda h, i: (0, 0)),
        ],
        out_specs=pl.BlockSpec((1, QBLK, DH), lambda h, i: (h, i, 0)),
        out_shape=jax.ShapeDtypeStruct((H, S, DH), jnp.float32),
    )(q, k, v, kp, W_perf)
    return jnp.transpose(out, (1, 0, 2)).reshape(B, S, H * DH)


# speedup vs baseline: 14.4234x; 1.0148x over previous
import jax
import jax.numpy as jnp
import numpy as np
from jax.experimental import pallas as pl
from jax.experimental.pallas import tpu as pltpu

B, H, S, DH = 1, 12, 2048, 64
M = 64
TOPK = 128
QBLK = 512


def _phi_block(x, w):
    proj = jax.lax.dot_general(
        x, w, (((1,), (0,)), ((), ())), preferred_element_type=jnp.float32)
    sq_norm = jnp.sum(x * x, axis=-1, keepdims=True) / 2.0
    stab = jnp.max(proj, axis=-1, keepdims=True)
    return jnp.exp(proj - sq_norm - stab) + 1e-6


def _kp_kernel(k_ref, w_ref, kp_ref):
    x = k_ref[0, 0] * (DH ** -0.25)
    kp_ref[0, 0] = _phi_block(x, w_ref[...])


def _bits16_to_bf(x_i32):
    return jax.lax.bitcast_convert_type(
        x_i32 << 16, jnp.float32).astype(jnp.bfloat16)


_CHUNK = 128


def _count_ge(x, cand, one, zero):
    acc = None
    for i in range(S // _CHUNK):
        m = jnp.where(x[i * _CHUNK:(i + 1) * _CHUNK] >= cand, one, zero)
        acc = m if acc is None else acc + m
    return jnp.sum(acc.astype(jnp.float32), axis=0, keepdims=True)


def _attn_kernel(q_ref, k_ref, v_ref, kp_ref, w_ref, out_ref):
    q = q_ref[0, 0]
    k = k_ref[0, 0]
    v = v_ref[0, 0]
    kp = kp_ref[0, 0]

    qp = _phi_block(q * (DH ** -0.25), w_ref[...])
    est_t = jax.lax.dot_general(
        kp, qp, (((1,), (1,)), ((), ())),
        preferred_element_type=jnp.float32)

    bits = jax.lax.bitcast_convert_type(est_t, jnp.int32)
    one_bf = jnp.bfloat16(1.0)
    zero_bf = jnp.bfloat16(0.0)
    ktop = jnp.float32(TOPK)

    est_bf = jax.lax.bitcast_convert_type(
        bits & jnp.int32(-0x10000),
        jnp.float32).astype(jnp.bfloat16)
    lo16 = jnp.full((1, QBLK), 0x2E80, jnp.int32)
    hi16b = jnp.full((1, QBLK), 0x4281, jnp.int32)
    for _ in range(13):
        mid = (lo16 + hi16b) >> 1
        cnt = _count_ge(est_bf, _bits16_to_bf(mid), one_bf, zero_bf)
        ge = cnt >= ktop
        lo16 = jnp.where(ge, mid, lo16)
        hi16b = jnp.where(ge, hi16b, mid)
    acc16 = lo16

    cnt_gt = _count_ge(est_bf, _bits16_to_bf(acc16 + 1), one_bf, zero_bf)
    k2 = ktop - cnt_gt

    in_bucket = est_bf == _bits16_to_bf(acc16)
    w_pat = ((bits & jnp.int32(0xFFFC)) >> 2) + jnp.int32(0x2000)
    w_bf = jnp.where(in_bucket, _bits16_to_bf(w_pat), zero_bf)
    accy = jnp.zeros((1, QBLK), jnp.int32)
    for b in range(13, -1, -1):
        candy = accy | jnp.int32(1 << b)
        cnt = _count_ge(w_bf, _bits16_to_bf(candy + jnp.int32(0x2000)),
                        one_bf, zero_bf)
        accy = jnp.where(cnt >= k2, candy, accy)

    acc = (acc16 << 16) | (accy << 2)
    for b in (1, 0):
        cand = acc | jnp.int32(1 << b)
        cnt = _count_ge(bits, cand, jnp.float32(1.0), jnp.float32(0.0))
        acc = jnp.where(cnt >= ktop, cand, acc)

    scores_t = jax.lax.dot_general(
        k, q, (((1,), (1,)), ((), ())),
        preferred_element_type=jnp.float32) * (1.0 / np.sqrt(DH))
    ms = jnp.where(bits >= acc, scores_t, jnp.float32(-1e9))
    mx = jnp.max(ms, axis=0, keepdims=True)
    p = jnp.exp(ms - mx)
    v_aug = jnp.concatenate([v, jnp.ones((S, 1), jnp.float32)], axis=1)
    ctx = jax.lax.dot_general(
        p, v_aug, (((0,), (0,)), ((), ())),
        preferred_element_type=jnp.float32)
    out_ref[0] = ctx[:, :DH] * (1.0 / ctx[:, DH:DH + 1])


@jax.jit
def kernel(q, k, v, W_perf):
    kp = pl.pallas_call(
        _kp_kernel,
        grid=(H,),
        in_specs=[
            pl.BlockSpec((1, 1, S, DH), lambda h: (0, h, 0, 0)),
            pl.BlockSpec((DH, M), lambda h: (0, 0)),
        ],
        out_specs=pl.BlockSpec((1, 1, S, M), lambda h: (0, h, 0, 0)),
        out_shape=jax.ShapeDtypeStruct((B, H, S, M), jnp.float32),
    )(k, W_perf)

    out = pl.pallas_call(
        _attn_kernel,
        grid=(H, S // QBLK),
        in_specs=[
            pl.BlockSpec((1, 1, QBLK, DH), lambda h, i: (0, h, i, 0)),
            pl.BlockSpec((1, 1, S, DH), lambda h, i: (0, h, 0, 0)),
            pl.BlockSpec((1, 1, S, DH), lambda h, i: (0, h, 0, 0)),
            pl.BlockSpec((1, 1, S, M), lambda h, i: (0, h, 0, 0)),
            pl.BlockSpec((DH, M), lambda h, i: (0, 0)),
        ],
        out_specs=pl.BlockSpec((1, QBLK, DH), lambda h, i: (h, i, 0)),
        out_shape=jax.ShapeDtypeStruct((H, S, DH), jnp.float32),
    )(q, k, v, kp, W_perf)
    return jnp.transpose(out, (1, 0, 2)).reshape(B, S, H * DH)
